# Initial kernel scaffold; baseline (speedup 1.0000x reference)
import jax
import jax.numpy as jnp
from jax.experimental import pallas as pl


def kernel(vectors, attributes):
    raise NotImplementedError("write your pallas kernel here")


# speedup vs baseline: 1.0014x; 1.0014x over previous
import jax
import jax.numpy as jnp
from jax.experimental import pallas as pl
from jax.experimental.pallas import tpu as pltpu


def _body(attrs_ref, vt_ref, out_ref, *, block_i: int):
    ai = pl.program_id(0)
    i = pl.program_id(1)
    attr = attrs_ref[ai]
    col = vt_ref[pl.ds(attr, 1), :]
    rows = vt_ref[pl.ds(attr, 1), pl.ds(i * block_i, block_i)]
    out_ref[0, :, :] = jnp.abs(rows[0][:, None] - col)


def kernel(vectors, attributes):
    n, f = vectors.shape
    a = attributes.shape[0]
    vt = vectors.T
    block_i = 512
    grid = (a, n // block_i)

    import functools
    body = functools.partial(_body, block_i=block_i)
    out = pl.pallas_call(
        body,
        grid=grid,
        in_specs=[
            pl.BlockSpec(memory_space=pltpu.SMEM),
            pl.BlockSpec((f, n), lambda ai, i: (0, 0)),
        ],
        out_specs=pl.BlockSpec((1, block_i, n), lambda ai, i: (ai, i, 0)),
        out_shape=jax.ShapeDtypeStruct((a, n, n), jnp.float32),
        compiler_params=pltpu.CompilerParams(
            dimension_semantics=("parallel", "parallel"),
        ),
    )(attributes.astype(jnp.int32), vt)
    return out
